# baseline (speedup 1.0000x reference)
import functools

import jax
import jax.numpy as jnp
from jax import lax
from jax.experimental import pallas as pl
from jax.experimental.pallas import tpu as pltpu
from jax.experimental.pallas import tpu_sc as plsc

_HIDDEN = 4096
_E = 64
_TOP_K = 8
_N_GROUP = 8
_TOPK_GROUP = 4
_SCALE = 2.5
_EPG = _E // _N_GROUP

_NC = 2
_NS = 16
_NW = _NC * _NS
_L = 16

_NEG = float("-inf")


def _mm_body(xa_ref, xb_ref, w_ref, b_ref, ca_ref, cb_ref):
    w = w_ref[...]
    b = b_ref[...]
    la = jax.lax.dot_general(
        w, xa_ref[...], (((1,), (1,)), ((), ())),
        preferred_element_type=jnp.float32)
    ca_ref[...] = jax.nn.sigmoid(la) + b
    lb = jax.lax.dot_general(
        w, xb_ref[...], (((1,), (1,)), ((), ())),
        preferred_element_type=jnp.float32)
    cb_ref[...] = jax.nn.sigmoid(lb) + b


def _mm(x, weight, bias):
    t = x.shape[0]
    bt = 512
    th = t // 2
    xa = x[:th]
    xb = x[th:]
    ca, cbk = pl.pallas_call(
        _mm_body,
        grid=(th // bt,),
        in_specs=[
            pl.BlockSpec((bt, _HIDDEN), lambda i: (i, 0)),
            pl.BlockSpec((bt, _HIDDEN), lambda i: (i, 0)),
            pl.BlockSpec((_E, _HIDDEN), lambda i: (0, 0)),
            pl.BlockSpec((_E, 1), lambda i: (0, 0)),
        ],
        out_specs=[
            pl.BlockSpec((_E, bt), lambda i: (0, i)),
            pl.BlockSpec((_E, bt), lambda i: (0, i)),
        ],
        out_shape=[
            jax.ShapeDtypeStruct((_E, th), jnp.float32),
            jax.ShapeDtypeStruct((_E, th), jnp.float32),
        ],
    )(xa, xb, weight, bias.reshape(_E, 1))
    return jnp.concatenate([ca, cbk], axis=1)


def _comb(a, b):
    take_a = a[0] >= b[0]
    return jnp.maximum(a[0], b[0]), jnp.where(take_a, a[1], b[1])


def _argmax_tree(nodes):
    while len(nodes) > 1:
        nodes = [_comb(nodes[i], nodes[i + 1])
                 for i in range(0, len(nodes), 2)]
    return nodes[0]


def _sc_route(choice_t, bias):
    t = choice_t.shape[1]
    tw = t // _NW
    nchunk = tw // _L

    mesh = plsc.VectorSubcoreMesh(
        core_axis_name="c", subcore_axis_name="s",
        num_cores=_NC, num_subcores=_NS)

    @functools.partial(
        pl.kernel,
        out_type=(jax.ShapeDtypeStruct((t * _TOP_K,), jnp.int32),
                  jax.ShapeDtypeStruct((t * _TOP_K,), jnp.float32)),
        mesh=mesh,
        compiler_params=pltpu.CompilerParams(needs_layout_passes=False),
        scratch_types=[
            pltpu.VMEM((_E, tw), jnp.float32),
            pltpu.VMEM((_E * _L,), jnp.float32),
            pltpu.VMEM((_E,), jnp.float32),
            pltpu.VMEM((tw * _TOP_K,), jnp.int32),
            pltpu.VMEM((tw * _TOP_K,), jnp.float32),
        ],
    )
    def route(choice_hbm, bias_hbm, idx_hbm, w_hbm, cv, cb, bv, ist, wst):
        wid = lax.axis_index("s") * _NC + lax.axis_index("c")
        base = wid * tw
        pltpu.sync_copy(choice_hbm.at[:, pl.ds(base, tw)], cv)
        pltpu.sync_copy(bias_hbm, bv)

        lane = lax.iota(jnp.int32, _L)
        neg = jnp.full((_L,), _NEG, jnp.float32)

        def chunk(j, _):
            col = j * _L
            c = [cv[e, pl.ds(col, _L)] for e in range(_E)]

            gs = []
            for g in range(_N_GROUP):
                m1 = c[g * _EPG]
                m2 = neg
                for e in range(g * _EPG + 1, (g + 1) * _EPG):
                    m2 = jnp.maximum(m2, jnp.minimum(m1, c[e]))
                    m1 = jnp.maximum(m1, c[e])
                gs.append(m1 + m2)

            rank = [jnp.zeros((_L,), jnp.int32) for _ in range(_N_GROUP)]
            for a in range(_N_GROUP):
                for b in range(a + 1, _N_GROUP):
                    rank[b] = rank[b] + (gs[a] >= gs[b]).astype(jnp.int32)
                    rank[a] = rank[a] + (gs[b] > gs[a]).astype(jnp.int32)
            gmask = [rank[g] < _TOPK_GROUP for g in range(_N_GROUP)]

            cm = []
            for e in range(_E):
                v = jnp.where(gmask[e // _EPG], c[e], 0.0)
                cb[pl.ds(e * _L, _L)] = v
                cm.append(v)

            tok8 = (lane + col) * _TOP_K
            wsum = jnp.zeros((_L,), jnp.float32)
            wks = []
            for k in range(_TOP_K):
                if k == 0:
                    vals = cm
                else:
                    vals = [cb[pl.ds(e * _L, _L)] for e in range(_E)]
                nodes = [(vals[e], jnp.full((_L,), e, jnp.int32))
                         for e in range(_E)]
                v, am = _argmax_tree(nodes)
                plsc.store_scatter(cb, [am * _L + lane], neg)
                wk = v - plsc.load_gather(bv, [am])
                plsc.store_scatter(ist, [tok8 + k], am)
                wks.append(wk)
                wsum = wsum + wk
            scale = _SCALE / (wsum + 1e-20)
            for k in range(_TOP_K):
                plsc.store_scatter(wst, [tok8 + k], wks[k] * scale)
            return 0

        lax.fori_loop(0, nchunk, chunk, 0)

        pltpu.sync_copy(ist, idx_hbm.at[pl.ds(base * _TOP_K, tw * _TOP_K)])
        pltpu.sync_copy(wst, w_hbm.at[pl.ds(base * _TOP_K, tw * _TOP_K)])

    return route(choice_t, bias)


def kernel(hidden_states, weight, e_score_correction_bias):
    bsz, seq_len, h = hidden_states.shape
    x = hidden_states.reshape(-1, h).astype(jnp.float32)
    t = x.shape[0]
    w32 = weight.astype(jnp.float32)
    bias = e_score_correction_bias.astype(jnp.float32)
    choice_t = _mm(x, w32, bias)
    idx_flat, w_flat = _sc_route(choice_t, bias)
    return (idx_flat.reshape(t, _TOP_K), w_flat.reshape(t, _TOP_K))


# speedup vs baseline: 1.7862x; 1.7862x over previous
import functools

import jax
import jax.numpy as jnp
from jax import lax
from jax.experimental import pallas as pl
from jax.experimental.pallas import tpu as pltpu
from jax.experimental.pallas import tpu_sc as plsc

_HIDDEN = 4096
_E = 64
_TOP_K = 8
_N_GROUP = 8
_TOPK_GROUP = 4
_SCALE = 2.5
_EPG = _E // _N_GROUP

_NC = 2
_NS = 16
_NW = _NC * _NS
_L = 16

_NEG = float("-inf")


def _mm_body(x_ref, w_ref, b_ref, c_ref):
    logits = jax.lax.dot_general(
        w_ref[...], x_ref[...], (((1,), (1,)), ((), ())),
        preferred_element_type=jnp.float32)
    c_ref[...] = jax.nn.sigmoid(logits) + b_ref[...]


def _mm(x, weight, bias):
    t = x.shape[0]
    bt = 1024
    return pl.pallas_call(
        _mm_body,
        grid=(t // bt,),
        in_specs=[
            pl.BlockSpec((bt, _HIDDEN), lambda i: (i, 0)),
            pl.BlockSpec((_E, _HIDDEN), lambda i: (0, 0)),
            pl.BlockSpec((_E, 1), lambda i: (0, 0)),
        ],
        out_specs=pl.BlockSpec((_E, bt), lambda i: (0, i)),
        out_shape=jax.ShapeDtypeStruct((_E, t), jnp.float32),
    )(x, weight, bias.reshape(_E, 1))


def _comb(a, b):
    take_a = a[0] >= b[0]
    return jnp.maximum(a[0], b[0]), jnp.where(take_a, a[1], b[1])


def _argmax_tree(nodes):
    while len(nodes) > 1:
        nodes = [_comb(nodes[i], nodes[i + 1])
                 for i in range(0, len(nodes), 2)]
    return nodes[0]


def _sc_route(choice_t, bias):
    t = choice_t.shape[1]
    tw = t // _NW
    nchunk = tw // _L

    mesh = plsc.VectorSubcoreMesh(
        core_axis_name="c", subcore_axis_name="s",
        num_cores=_NC, num_subcores=_NS)

    @functools.partial(
        pl.kernel,
        out_type=(jax.ShapeDtypeStruct((t * _TOP_K,), jnp.int32),
                  jax.ShapeDtypeStruct((t * _TOP_K,), jnp.float32)),
        mesh=mesh,
        compiler_params=pltpu.CompilerParams(needs_layout_passes=False),
        scratch_types=[
            pltpu.VMEM((_E, tw), jnp.float32),
            pltpu.VMEM((2 * _E * _L,), jnp.float32),
            pltpu.VMEM((_E,), jnp.float32),
            pltpu.VMEM((tw * _TOP_K,), jnp.int32),
            pltpu.VMEM((tw * _TOP_K,), jnp.float32),
        ],
    )
    def route(choice_hbm, bias_hbm, idx_hbm, w_hbm, cv, cb, bv, ist, wst):
        wid = lax.axis_index("s") * _NC + lax.axis_index("c")
        base = wid * tw
        pltpu.sync_copy(choice_hbm.at[:, pl.ds(base, tw)], cv)
        pltpu.sync_copy(bias_hbm, bv)

        lane = lax.iota(jnp.int32, _L)
        neg = jnp.full((_L,), _NEG, jnp.float32)

        @plsc.parallel_loop(0, nchunk, 1, unroll=2)
        def chunk(j):
            col = j * _L
            off = (j & 1) * (_E * _L)
            c = [cv[e, pl.ds(col, _L)] for e in range(_E)]

            gs = []
            for g in range(_N_GROUP):
                m1 = c[g * _EPG]
                m2 = neg
                for e in range(g * _EPG + 1, (g + 1) * _EPG):
                    m2 = jnp.maximum(m2, jnp.minimum(m1, c[e]))
                    m1 = jnp.maximum(m1, c[e])
                gs.append(m1 + m2)

            rank = [jnp.zeros((_L,), jnp.int32) for _ in range(_N_GROUP)]
            for a in range(_N_GROUP):
                for b in range(a + 1, _N_GROUP):
                    rank[b] = rank[b] + (gs[a] >= gs[b]).astype(jnp.int32)
                    rank[a] = rank[a] + (gs[b] > gs[a]).astype(jnp.int32)
            gmask = [rank[g] < _TOPK_GROUP for g in range(_N_GROUP)]

            cm = []
            for e in range(_E):
                v = jnp.where(gmask[e // _EPG], c[e], 0.0)
                cb[pl.ds(off + e * _L, _L)] = v
                cm.append(v)

            tok8 = (lane + col) * _TOP_K
            wsum = jnp.zeros((_L,), jnp.float32)
            wks = []
            for k in range(_TOP_K):
                if k == 0:
                    vals = cm
                else:
                    vals = [cb[pl.ds(off + e * _L, _L)] for e in range(_E)]
                nodes = [(vals[e], jnp.full((_L,), e, jnp.int32))
                         for e in range(_E)]
                v, am = _argmax_tree(nodes)
                plsc.store_scatter(cb, [off + am * _L + lane], neg)
                wk = v - plsc.load_gather(bv, [am])
                plsc.store_scatter(ist, [tok8 + k], am)
                wks.append(wk)
                wsum = wsum + wk
            scale = _SCALE / (wsum + 1e-20)
            for k in range(_TOP_K):
                plsc.store_scatter(wst, [tok8 + k], wks[k] * scale)

        pltpu.sync_copy(ist, idx_hbm.at[pl.ds(base * _TOP_K, tw * _TOP_K)])
        pltpu.sync_copy(wst, w_hbm.at[pl.ds(base * _TOP_K, tw * _TOP_K)])

    return route(choice_t, bias)


def kernel(hidden_states, weight, e_score_correction_bias):
    bsz, seq_len, h = hidden_states.shape
    x = hidden_states.reshape(-1, h).astype(jnp.float32)
    t = x.shape[0]
    w32 = weight.astype(jnp.float32)
    bias = e_score_correction_bias.astype(jnp.float32)
    choice_t = _mm(x, w32, bias)
    idx_flat, w_flat = _sc_route(choice_t, bias)
    return (idx_flat.reshape(t, _TOP_K), w_flat.reshape(t, _TOP_K))


# speedup vs baseline: 2.1005x; 1.1760x over previous
import functools

import jax
import jax.numpy as jnp
from jax import lax
from jax.experimental import pallas as pl
from jax.experimental.pallas import tpu as pltpu
from jax.experimental.pallas import tpu_sc as plsc

_HIDDEN = 4096
_E = 64
_TOP_K = 8
_N_GROUP = 8
_TOPK_GROUP = 4
_SCALE = 2.5
_EPG = _E // _N_GROUP

_NC = 2
_NS = 16
_NW = _NC * _NS
_L = 16

_NEG = float("-inf")


def _mm_body(x_ref, w_ref, b_ref, c_ref):
    logits = jax.lax.dot_general(
        w_ref[...], x_ref[...], (((1,), (1,)), ((), ())),
        preferred_element_type=jnp.float32)
    c_ref[...] = jax.nn.sigmoid(logits) + b_ref[...]


def _mm(x, weight, bias):
    t = x.shape[0]
    bt = 1024
    return pl.pallas_call(
        _mm_body,
        grid=(t // bt,),
        in_specs=[
            pl.BlockSpec((bt, _HIDDEN), lambda i: (i, 0)),
            pl.BlockSpec((_E, _HIDDEN), lambda i: (0, 0)),
            pl.BlockSpec((_E, 1), lambda i: (0, 0)),
        ],
        out_specs=pl.BlockSpec((_E, bt), lambda i: (0, i)),
        out_shape=jax.ShapeDtypeStruct((_E, t), jnp.float32),
    )(x, weight, bias.reshape(_E, 1))


def _comb(a, b):
    take_a = a[0] >= b[0]
    return jnp.maximum(a[0], b[0]), jnp.where(take_a, a[1], b[1])


def _argmax_tree(nodes):
    while len(nodes) > 1:
        nodes = [_comb(nodes[i], nodes[i + 1])
                 for i in range(0, len(nodes), 2)]
    return nodes[0]


def _sc_route(choice_t, bias):
    t = choice_t.shape[1]
    tw = t // _NW
    nchunk = tw // _L

    mesh = plsc.VectorSubcoreMesh(
        core_axis_name="c", subcore_axis_name="s",
        num_cores=_NC, num_subcores=_NS)

    @functools.partial(
        pl.kernel,
        out_type=(jax.ShapeDtypeStruct((t * _TOP_K,), jnp.int32),
                  jax.ShapeDtypeStruct((t * _TOP_K,), jnp.float32)),
        mesh=mesh,
        compiler_params=pltpu.CompilerParams(needs_layout_passes=False),
        scratch_types=[
            pltpu.VMEM((_E, tw), jnp.float32),
            pltpu.VMEM((_E * _L,), jnp.float32),
            pltpu.VMEM((_E,), jnp.float32),
            pltpu.VMEM((tw * _TOP_K,), jnp.int32),
            pltpu.VMEM((tw * _TOP_K,), jnp.float32),
        ],
    )
    def route(choice_hbm, bias_hbm, idx_hbm, w_hbm, cv, cb, bv, ist, wst):
        wid = lax.axis_index("s") * _NC + lax.axis_index("c")
        base = wid * tw
        pltpu.sync_copy(choice_hbm.at[:, pl.ds(base, tw)], cv)
        pltpu.sync_copy(bias_hbm, bv)

        lane = lax.iota(jnp.int32, _L)
        neg = jnp.full((_L,), _NEG, jnp.float32)

        def chunk(j, _):
            col = j * _L
            c = [cv[e, pl.ds(col, _L)] for e in range(_E)]

            gs = []
            for g in range(_N_GROUP):
                m1 = c[g * _EPG]
                m2 = neg
                for e in range(g * _EPG + 1, (g + 1) * _EPG):
                    m2 = jnp.maximum(m2, jnp.minimum(m1, c[e]))
                    m1 = jnp.maximum(m1, c[e])
                gs.append(m1 + m2)

            rank = [jnp.zeros((_L,), jnp.int32) for _ in range(_N_GROUP)]
            for a in range(_N_GROUP):
                for b in range(a + 1, _N_GROUP):
                    rank[b] = rank[b] + (gs[a] >= gs[b]).astype(jnp.int32)
                    rank[a] = rank[a] + (gs[b] > gs[a]).astype(jnp.int32)
            gmask = [rank[g] < _TOPK_GROUP for g in range(_N_GROUP)]

            cm = []
            for e in range(_E):
                v = jnp.where(gmask[e // _EPG], c[e], 0.0)
                cb[pl.ds(e * _L, _L)] = v
                cm.append(v)

            tok8 = (lane + col) * _TOP_K
            wsum = jnp.zeros((_L,), jnp.float32)
            wks = []
            for k in range(_TOP_K):
                if k == 0:
                    vals = cm
                else:
                    vals = [cb[pl.ds(e * _L, _L)] for e in range(_E)]
                nodes = [(vals[e], jnp.full((_L,), e, jnp.int32))
                         for e in range(_E)]
                v, am = _argmax_tree(nodes)
                plsc.store_scatter(cb, [am * _L + lane], neg)
                wk = v - plsc.load_gather(bv, [am])
                plsc.store_scatter(ist, [tok8 + k], am)
                wks.append(wk)
                wsum = wsum + wk
            scale = _SCALE / (wsum + 1e-20)
            for k in range(_TOP_K):
                plsc.store_scatter(wst, [tok8 + k], wks[k] * scale)
            return 0

        lax.fori_loop(0, nchunk, chunk, 0)

        pltpu.sync_copy(ist, idx_hbm.at[pl.ds(base * _TOP_K, tw * _TOP_K)])
        pltpu.sync_copy(wst, w_hbm.at[pl.ds(base * _TOP_K, tw * _TOP_K)])

    return route(choice_t, bias)


def kernel(hidden_states, weight, e_score_correction_bias):
    bsz, seq_len, h = hidden_states.shape
    x = hidden_states.reshape(-1, h).astype(jnp.float32)
    t = x.shape[0]
    w32 = weight.astype(jnp.float32)
    bias = e_score_correction_bias.astype(jnp.float32)
    choice_t = _mm(x, w32, bias)
    idx_flat, w_flat = _sc_route(choice_t, bias)
    return (idx_flat.reshape(t, _TOP_K), w_flat.reshape(t, _TOP_K))
